# Initial kernel scaffold; baseline (speedup 1.0000x reference)
import jax
import jax.numpy as jnp
from jax.experimental import pallas as pl


def kernel(betas, t_int):
    raise NotImplementedError("write your pallas kernel here")


# speedup vs baseline: 3.3107x; 3.3107x over previous
import functools

import jax
import jax.numpy as jnp
from jax import lax
from jax.experimental import pallas as pl
from jax.experimental.pallas import tpu as pltpu
from jax.experimental.pallas import tpu_sc as plsc

_BATCH = 16384

_info = plsc.get_sparse_core_info()
_NC = _info.num_cores
_NS = _info.num_subcores
_NW = _NC * _NS
_B_PER_W = _BATCH // _NW
_CHUNK = 128
_NCHUNK = _B_PER_W // _CHUNK


def _sc_gather_kernel(betas_hbm, idx_hbm, out_hbm, idx_v, out_v, sem):
    wid = lax.axis_index("s") * _NC + lax.axis_index("c")
    pltpu.sync_copy(idx_hbm.at[wid], idx_v)
    copies = [
        pltpu.async_copy(betas_hbm.at[idx_v.at[j]], out_v.at[j], sem)
        for j in range(_NCHUNK)
    ]
    for c in copies:
        c.wait()
    pltpu.sync_copy(out_v, out_hbm.at[wid])


@jax.jit
def kernel(betas, t_int):
    idx = t_int.astype(jnp.int32).reshape(_NW, _NCHUNK, _CHUNK)
    mesh = plsc.VectorSubcoreMesh(core_axis_name="c", subcore_axis_name="s")
    run = functools.partial(
        pl.kernel,
        mesh=mesh,
        out_type=jax.ShapeDtypeStruct((_NW, _NCHUNK, _CHUNK), jnp.float32),
        scratch_types=[
            pltpu.VMEM((_NCHUNK, _CHUNK), jnp.int32),
            pltpu.VMEM((_NCHUNK, _CHUNK), jnp.float32),
            pltpu.SemaphoreType.DMA,
        ],
    )(_sc_gather_kernel)
    return run(betas, idx).reshape(_BATCH)
